# baseline (speedup 1.0000x reference)
import functools

import jax
import jax.numpy as jnp
from jax import lax
from jax.experimental import pallas as pl
from jax.experimental.pallas import tpu as pltpu
from jax.experimental.pallas import tpu_sc as plsc

_VOCAB = 100000
_D = 128
_SEQ = 200
_BATCH = 4096

_NC = 2
_NS = 16
_LANES = 16
_NW = _NC * _NS
_BPW = _BATCH // _NW
_GROUPS = _BPW // _LANES

_G_STEPS = 10
_ROWS_PER_BLOCK = _VOCAB // _G_STEPS
_VOCAB_PADDED = ((_VOCAB + 127) // 128) * 128


def _table_proj_body(emb_ref, w_ref, b_ref, out_ref):
    g = pl.program_id(0)
    e = emb_ref[0]
    s = jax.lax.dot_general(
        w_ref[...], e, (((1,), (1,)), ((), ())),
        preferred_element_type=jnp.float32,
    )
    s = (s + b_ref[0]) * (1.0 / _SEQ)
    for k in range(_G_STEPS):
        @pl.when(g == k)
        def _store(k=k):
            out_ref[0, :, k * _ROWS_PER_BLOCK:(k + 1) * _ROWS_PER_BLOCK] = s


def _project_table(emb3, fc_w, fc_b):
    return pl.pallas_call(
        _table_proj_body,
        grid=(_G_STEPS,),
        in_specs=[
            pl.BlockSpec((1, _ROWS_PER_BLOCK, _D), lambda g: (g, 0, 0)),
            pl.BlockSpec((1, _D), lambda g: (0, 0)),
            pl.BlockSpec(memory_space=pltpu.SMEM),
        ],
        out_specs=pl.BlockSpec((1, 1, _VOCAB_PADDED), lambda g: (0, 0, 0)),
        out_shape=jax.ShapeDtypeStruct((1, 1, _VOCAB_PADDED), jnp.float32),
    )(emb3, fc_w, fc_b)


_SC_MESH = plsc.VectorSubcoreMesh(
    core_axis_name="c", subcore_axis_name="s", num_cores=_NC, num_subcores=_NS
)


@functools.partial(
    pl.kernel,
    mesh=_SC_MESH,
    compiler_params=pltpu.CompilerParams(needs_layout_passes=False),
    out_type=jax.ShapeDtypeStruct((_BATCH,), jnp.float32),
    scratch_types=[
        pltpu.VMEM((_SEQ, _BPW), jnp.int32),
        pltpu.VMEM((_VOCAB_PADDED,), jnp.float32),
        pltpu.VMEM_SHARED((_VOCAB_PADDED // 2,), jnp.float32),
        pltpu.VMEM((_BPW,), jnp.float32),
        pltpu.SemaphoreType.DMA,
    ],
)
def _sc_pool(x_hbm, p_hbm, out_hbm, idx_v, p_v, p_sh, out_v, sem):
    _HALF = _VOCAB_PADDED // 2
    sid = lax.axis_index("s")
    wid = sid * _NC + lax.axis_index("c")
    base = wid * _BPW
    with jax.named_scope("dma_start"):
        cp_idx = pltpu.async_copy(x_hbm.at[:, pl.ds(base, _BPW)], idx_v, sem)

    with jax.named_scope("dma_wait"):
        @pl.when(sid == 0)
        def _stage_p0():
            pltpu.sync_copy(p_hbm.at[0, 0, pl.ds(0, _HALF)], p_sh)

        plsc.subcore_barrier()
        pltpu.sync_copy(p_sh, p_v.at[pl.ds(0, _HALF)])
        plsc.subcore_barrier()

        @pl.when(sid == 0)
        def _stage_p1():
            pltpu.sync_copy(p_hbm.at[0, 0, pl.ds(_HALF, _HALF)], p_sh)

        plsc.subcore_barrier()
        pltpu.sync_copy(p_sh, p_v.at[pl.ds(_HALF, _HALF)])
        cp_idx.wait()

    zero = jnp.zeros((_LANES,), jnp.float32)

    def body(l, accs):
        new = []
        for g in range(_GROUPS):
            idx = idx_v[l, pl.ds(g * _LANES, _LANES)]
            new.append(accs[g] + plsc.load_gather(p_v, [idx]))
        return tuple(new)

    with jax.named_scope("gather_loop"):
        accs = lax.fori_loop(0, _SEQ, body, (zero,) * _GROUPS)
    with jax.named_scope("writeback"):
        for g in range(_GROUPS):
            out_v[pl.ds(g * _LANES, _LANES)] = accs[g]
        pltpu.sync_copy(out_v, out_hbm.at[pl.ds(base, _BPW)])


def kernel(x, emb_table, fc_w, fc_b):
    emb3 = emb_table.reshape(_G_STEPS, _ROWS_PER_BLOCK, _D)
    p3 = _project_table(emb3, fc_w, fc_b)
    return _sc_pool(x, p3)


# speedup vs baseline: 1.0513x; 1.0513x over previous
import functools

import jax
import jax.numpy as jnp
from jax import lax
from jax.experimental import pallas as pl
from jax.experimental.pallas import tpu as pltpu
from jax.experimental.pallas import tpu_sc as plsc

_VOCAB = 100000
_D = 128
_SEQ = 200
_BATCH = 4096

_NC = 2
_NS = 16
_LANES = 16
_NW = _NC * _NS
_BPW = _BATCH // _NW
_GROUPS = _BPW // _LANES

_G_STEPS = 5
_ROWS_PER_BLOCK = _VOCAB // _G_STEPS
_VOCAB_PADDED = ((_VOCAB + 127) // 128) * 128


def _table_proj_body(emb_ref, w_ref, b_ref, out_ref):
    g = pl.program_id(0)
    e = emb_ref[0]
    s = jax.lax.dot_general(
        w_ref[...], e, (((1,), (1,)), ((), ())),
        preferred_element_type=jnp.float32,
    )
    s = (s + b_ref[0]) * (1.0 / _SEQ)
    for k in range(_G_STEPS):
        @pl.when(g == k)
        def _store(k=k):
            out_ref[0, :, k * _ROWS_PER_BLOCK:(k + 1) * _ROWS_PER_BLOCK] = s


def _project_table(emb3, fc_w, fc_b):
    return pl.pallas_call(
        _table_proj_body,
        grid=(_G_STEPS,),
        in_specs=[
            pl.BlockSpec((1, _ROWS_PER_BLOCK, _D), lambda g: (g, 0, 0)),
            pl.BlockSpec((1, _D), lambda g: (0, 0)),
            pl.BlockSpec(memory_space=pltpu.SMEM),
        ],
        out_specs=pl.BlockSpec((1, 1, _VOCAB_PADDED), lambda g: (0, 0, 0)),
        out_shape=jax.ShapeDtypeStruct((1, 1, _VOCAB_PADDED), jnp.float32),
    )(emb3, fc_w, fc_b)


_SC_MESH = plsc.VectorSubcoreMesh(
    core_axis_name="c", subcore_axis_name="s", num_cores=_NC, num_subcores=_NS
)


@functools.partial(
    pl.kernel,
    mesh=_SC_MESH,
    compiler_params=pltpu.CompilerParams(needs_layout_passes=False),
    out_type=jax.ShapeDtypeStruct((_BATCH,), jnp.float32),
    scratch_types=[
        pltpu.VMEM((_SEQ, _BPW), jnp.int32),
        pltpu.VMEM((_VOCAB_PADDED,), jnp.float32),
        pltpu.VMEM_SHARED((_VOCAB_PADDED // 2,), jnp.float32),
        pltpu.VMEM((_BPW,), jnp.float32),
        pltpu.SemaphoreType.DMA,
    ],
)
def _sc_pool(x_hbm, p_hbm, out_hbm, idx_v, p_v, p_sh, out_v, sem):
    _HALF = _VOCAB_PADDED // 2
    sid = lax.axis_index("s")
    wid = sid * _NC + lax.axis_index("c")
    base = wid * _BPW
    with jax.named_scope("dma_start"):
        cp_idx = pltpu.async_copy(x_hbm.at[:, pl.ds(base, _BPW)], idx_v, sem)

    with jax.named_scope("dma_wait"):
        @pl.when(sid == 0)
        def _stage_p0():
            pltpu.sync_copy(p_hbm.at[0, 0, pl.ds(0, _HALF)], p_sh)

        _Q = _HALF // 2
        plsc.subcore_barrier()
        f0 = pltpu.async_copy(p_sh.at[pl.ds(0, _Q)], p_v.at[pl.ds(0, _Q)], sem)
        f1 = pltpu.async_copy(p_sh.at[pl.ds(_Q, _Q)], p_v.at[pl.ds(_Q, _Q)], sem)
        f0.wait()
        f1.wait()
        plsc.subcore_barrier()

        @pl.when(sid == 0)
        def _stage_p1():
            pltpu.sync_copy(p_hbm.at[0, 0, pl.ds(_HALF, _HALF)], p_sh)

        plsc.subcore_barrier()
        f2 = pltpu.async_copy(
            p_sh.at[pl.ds(0, _Q)], p_v.at[pl.ds(_HALF, _Q)], sem
        )
        f3 = pltpu.async_copy(
            p_sh.at[pl.ds(_Q, _Q)], p_v.at[pl.ds(_HALF + _Q, _Q)], sem
        )
        f2.wait()
        f3.wait()
        cp_idx.wait()

    zero = jnp.zeros((_LANES,), jnp.float32)

    def body(l, accs):
        new = []
        for g in range(_GROUPS):
            idx = idx_v[l, pl.ds(g * _LANES, _LANES)]
            new.append(accs[g] + plsc.load_gather(p_v, [idx]))
        return tuple(new)

    with jax.named_scope("gather_loop"):
        accs = lax.fori_loop(0, _SEQ, body, (zero,) * _GROUPS)
    with jax.named_scope("writeback"):
        for g in range(_GROUPS):
            out_v[pl.ds(g * _LANES, _LANES)] = accs[g]
        pltpu.sync_copy(out_v, out_hbm.at[pl.ds(base, _BPW)])


def kernel(x, emb_table, fc_w, fc_b):
    emb3 = emb_table.reshape(_G_STEPS, _ROWS_PER_BLOCK, _D)
    p3 = _project_table(emb3, fc_w, fc_b)
    return _sc_pool(x, p3)
